# baseline (speedup 1.0000x reference)
import functools

import jax
import jax.numpy as jnp
from jax import lax
from jax.experimental import pallas as pl
from jax.experimental.pallas import tpu as pltpu
from jax.experimental.pallas import tpu_sc as plsc

_N = 10000
_E = 320000
_D = 128
_G = 16

_NP = 10240
_NC = 2
_NS = 16
_NW = _NC * _NS
_C = 128
_CH = 80
_EP = _NW * _CH * _C
_RPS = _NP // _NS
_R = 4
_S0A, _S1A = 52, 108
_S0B, _S1B = 36, 124

_BLK = 1024
_NB = _NP // _BLK


def _mesh():
    return plsc.VectorSubcoreMesh(core_axis_name="c", subcore_axis_name="s",
                                  num_cores=_NC, num_subcores=_NS)


@functools.cache
def _deg_kernel(c0, c1):
    chm = max(c0, c1)

    @functools.partial(
        pl.kernel,
        out_type=jax.ShapeDtypeStruct((_NC, _NP, 16), jnp.float32),
        mesh=_mesh(),
        scratch_types=[
            pltpu.VMEM((chm, _C), jnp.int32),
            pltpu.VMEM((_C, 16), jnp.float32),
            pltpu.VMEM_SHARED((_NP, 16), jnp.float32),
            pltpu.SemaphoreType.DMA,
        ],
        compiler_params=pltpu.CompilerParams(use_tc_tiling_on_sc=False),
    )
    def deg_k(dst_hbm, ones_hbm, zeros_hbm, out_hbm, dst_v, ones_v, acc_sh, sem):
        cid = lax.axis_index("c")
        sid = lax.axis_index("s")
        wid = cid * _NS + sid
        r0 = sid * _RPS
        nch = jnp.where(cid == 0, c0, c1)
        pltpu.sync_copy(zeros_hbm.at[pl.ds(r0, _RPS)], acc_sh.at[pl.ds(r0, _RPS)])
        pltpu.sync_copy(dst_hbm.at[wid], dst_v)
        pltpu.sync_copy(ones_hbm, ones_v)
        plsc.subcore_barrier()

        def body(ch, carry):
            pltpu.sync_copy(ones_v, acc_sh.at[dst_v.at[ch]], add=True)
            return carry

        lax.fori_loop(0, nch, body, 0)
        plsc.subcore_barrier()
        pltpu.sync_copy(acc_sh.at[pl.ds(r0, _RPS)], out_hbm.at[cid, pl.ds(r0, _RPS)])

    return deg_k


@functools.cache
def _agg_kernel(d, c0, c1):
    chm = max(c0, c1)

    @functools.partial(
        pl.kernel,
        out_type=jax.ShapeDtypeStruct((_NC, _NP, d), jnp.float32),
        mesh=_mesh(),
        scratch_types=[
            pltpu.VMEM((chm, _C), jnp.int32),
            pltpu.VMEM((chm, _C), jnp.int32),
            pltpu.VMEM((_R, _C, d), jnp.float32),
            pltpu.VMEM_SHARED((_NP, d), jnp.float32),
            [pltpu.SemaphoreType.DMA] * _R,
            [pltpu.SemaphoreType.DMA] * _R,
        ],
        compiler_params=pltpu.CompilerParams(use_tc_tiling_on_sc=False),
    )
    def agg_k(hs_hbm, src_hbm, dst_hbm, zeros_hbm, out_hbm,
              src_v, dst_v, rows_v, acc_sh, gsem, ssem):
        cid = lax.axis_index("c")
        sid = lax.axis_index("s")
        wid = cid * _NS + sid
        r0 = sid * _RPS
        nch = jnp.where(cid == 0, c0, c1)
        pltpu.sync_copy(zeros_hbm.at[pl.ds(r0, _RPS)], acc_sh.at[pl.ds(r0, _RPS)])
        plsc.subcore_barrier()

        def gather(ch, b):
            return pltpu.async_copy(hs_hbm.at[src_v.at[ch]], rows_v.at[b], gsem[b])

        def scatter(ch, b):
            return pltpu.async_copy(rows_v.at[b], acc_sh.at[dst_v.at[ch]],
                                    ssem[b], add=True)

        pltpu.sync_copy(src_hbm.at[wid], src_v)
        pltpu.sync_copy(dst_hbm.at[wid], dst_v)
        for b in range(_R):
            gather(b, b)

        def body(i, carry):
            for b in range(_R):
                ch = i * _R + b
                pltpu.make_async_copy(hs_hbm.at[src_v.at[ch]],
                                      rows_v.at[b], gsem[b]).wait()
                scatter(ch, b)
                pb = b - 1 if b else _R - 1
                pch = ch - 1

                @pl.when(pch >= 0)
                def _():
                    pltpu.make_async_copy(
                        rows_v.at[pb],
                        acc_sh.at[dst_v.at[lax.max(pch, 0)]],
                        ssem[pb]).wait()

                    @pl.when(pch + _R < nch)
                    def _():
                        gather(pch + _R, pb)
            return carry

        lax.fori_loop(0, nch // _R, body, 0)
        pltpu.make_async_copy(rows_v.at[_R - 1],
                              acc_sh.at[dst_v.at[nch - 1]],
                              ssem[_R - 1]).wait()
        plsc.subcore_barrier()
        pltpu.sync_copy(acc_sh.at[pl.ds(r0, _RPS)], out_hbm.at[cid, pl.ds(r0, _RPS)])

    return agg_k


@functools.cache
def _tc1():
    def body(p0, p1, x, w, dinv_ref, hs_ref):
        deg = p0[:, 0:1] + p1[:, 0:1] + 1.0
        dinv = 1.0 / jnp.sqrt(deg)
        dinv_ref[...] = dinv
        hs_ref[...] = dinv * jnp.dot(x[...], w[...],
                                     preferred_element_type=jnp.float32)

    return pl.pallas_call(
        body,
        grid=(_NB,),
        in_specs=[
            pl.BlockSpec((_BLK, 16), lambda i: (i, 0)),
            pl.BlockSpec((_BLK, 16), lambda i: (i, 0)),
            pl.BlockSpec((_BLK, _D), lambda i: (i, 0)),
            pl.BlockSpec((_D, 16), lambda i: (0, 0)),
        ],
        out_specs=[
            pl.BlockSpec((_BLK, 1), lambda i: (i, 0)),
            pl.BlockSpec((_BLK, 16), lambda i: (i, 0)),
        ],
        out_shape=[
            jax.ShapeDtypeStruct((_NP, 1), jnp.float32),
            jax.ShapeDtypeStruct((_NP, 16), jnp.float32),
        ],
    )


@functools.cache
def _tc2(di, do):
    def body(p0, p1, hs, dinv, b, w, out_ref):
        i = pl.program_id(0)
        rid = lax.broadcasted_iota(jnp.int32, (_BLK, 1), 0) + i * _BLK
        dv = dinv[...]
        h = dv * (p0[...] + p1[...] + hs[...]) + b[...]
        h = jnp.where(rid < _N, jnp.maximum(h, 0.0), 0.0)
        out_ref[...] = dv * jnp.dot(h, w[...], preferred_element_type=jnp.float32)

    return pl.pallas_call(
        body,
        grid=(_NB,),
        in_specs=[
            pl.BlockSpec((_BLK, di), lambda i: (i, 0)),
            pl.BlockSpec((_BLK, di), lambda i: (i, 0)),
            pl.BlockSpec((_BLK, di), lambda i: (i, 0)),
            pl.BlockSpec((_BLK, 1), lambda i: (i, 0)),
            pl.BlockSpec((1, di), lambda i: (0, 0)),
            pl.BlockSpec((di, do), lambda i: (0, 0)),
        ],
        out_specs=pl.BlockSpec((_BLK, do), lambda i: (i, 0)),
        out_shape=jax.ShapeDtypeStruct((_NP, do), jnp.float32),
    )


@functools.cache
def _tc3():
    def body(p0, p1, hs, dinv, b, bat, wl1, bl1, wl2, bl2, out_ref, g_ref):
        i = pl.program_id(0)

        @pl.when(i == 0)
        def _init():
            g_ref[...] = jnp.full((_G, 64), -jnp.inf, jnp.float32)

        rid = lax.broadcasted_iota(jnp.int32, (_BLK, 1), 0) + i * _BLK
        h = dinv[...] * (p0[...] + p1[...] + hs[...]) + b[...]
        h = jnp.where(rid < _N, jnp.maximum(h, 0.0), -jnp.inf)
        bv = bat[...]
        parts = [jnp.max(jnp.where(bv == g, h, -jnp.inf), axis=0, keepdims=True)
                 for g in range(_G)]
        g_ref[...] = jnp.maximum(g_ref[...], jnp.concatenate(parts, axis=0))

        @pl.when(i == _NB - 1)
        def _finish():
            gg = g_ref[...]
            z = jnp.maximum(
                jnp.dot(gg, wl1[...], preferred_element_type=jnp.float32)
                + bl1[...], 0.0)
            o = jnp.dot(z, wl2[...], preferred_element_type=jnp.float32) + bl2[...]
            out_ref[...] = 1.0 / (1.0 + jnp.exp(-o))

    return pl.pallas_call(
        body,
        grid=(_NB,),
        in_specs=[
            pl.BlockSpec((_BLK, 64), lambda i: (i, 0)),
            pl.BlockSpec((_BLK, 64), lambda i: (i, 0)),
            pl.BlockSpec((_BLK, 64), lambda i: (i, 0)),
            pl.BlockSpec((_BLK, 1), lambda i: (i, 0)),
            pl.BlockSpec((1, 64), lambda i: (0, 0)),
            pl.BlockSpec((_BLK, 1), lambda i: (i, 0)),
            pl.BlockSpec((64, 256), lambda i: (0, 0)),
            pl.BlockSpec((1, 256), lambda i: (0, 0)),
            pl.BlockSpec((256, 10), lambda i: (0, 0)),
            pl.BlockSpec((1, 10), lambda i: (0, 0)),
        ],
        out_specs=pl.BlockSpec((_G, 10), lambda i: (0, 0)),
        out_shape=jax.ShapeDtypeStruct((_G, 10), jnp.float32),
        scratch_shapes=[pltpu.VMEM((_G, 64), jnp.float32)],
    )


def kernel(x, edge_index, batch, W1, b1, W2, b2, W3, b3, Wl1, bl1, Wl2, bl2):
    f32 = jnp.float32
    x_p = jnp.pad(x, ((0, _NP - _N), (0, 0)))
    ei = edge_index.astype(jnp.int32)
    pad_e = jnp.full((_EP - _E,), _N, jnp.int32)
    srcp = jnp.concatenate([ei[0], pad_e])
    dstp = jnp.concatenate([ei[1], pad_e])

    def layout(v, c0, c1):
        chm = max(c0, c1)
        m = _NS * c0 * _C
        a0 = v[:m].reshape(_NS, c0, _C)
        a1 = v[m:m + _NS * c1 * _C].reshape(_NS, c1, _C)
        a0 = jnp.pad(a0, ((0, 0), (0, chm - c0), (0, 0)), constant_values=_N)
        a1 = jnp.pad(a1, ((0, 0), (0, chm - c1), (0, 0)), constant_values=_N)
        return jnp.concatenate([a0, a1], axis=0)

    src_a = layout(srcp, _S0A, _S1A)
    dst_a = layout(dstp, _S0A, _S1A)
    src_b = layout(srcp, _S0B, _S1B)
    dst_b = layout(dstp, _S0B, _S1B)
    bat_p = jnp.concatenate(
        [batch.astype(jnp.int32), jnp.full((_NP - _N,), _G, jnp.int32)]
    ).reshape(_NP, 1)
    ones16 = jnp.ones((_C, 16), f32)
    z16 = jnp.zeros((_NP, 16), f32)
    z32 = jnp.zeros((_NP, 32), f32)
    z64 = jnp.zeros((_NP, 64), f32)

    degp = _deg_kernel(_S0A, _S1A)(dst_a, ones16, z16)
    dinv, hs1 = _tc1()(degp[0], degp[1], x_p, W1)
    p1 = _agg_kernel(16, _S0A, _S1A)(hs1, src_a, dst_a, z16)
    hs2 = _tc2(16, 32)(p1[0], p1[1], hs1, dinv, b1.reshape(1, -1), W2)
    p2 = _agg_kernel(32, _S0A, _S1A)(hs2, src_a, dst_a, z32)
    hs3 = _tc2(32, 64)(p2[0], p2[1], hs2, dinv, b2.reshape(1, -1), W3)
    p3 = _agg_kernel(64, _S0B, _S1B)(hs3, src_b, dst_b, z64)
    out = _tc3()(p3[0], p3[1], hs3, dinv, b3.reshape(1, -1), bat_p,
                 Wl1, bl1.reshape(1, -1), Wl2, bl2.reshape(1, -1))
    return out
